# baseline (speedup 1.0000x reference)
import functools

import jax
import jax.numpy as jnp
from jax import lax
from jax.experimental import pallas as pl
from jax.experimental.pallas import tpu as pltpu
from jax.experimental.pallas import tpu_sc as plsc

H = 720
W = 1280
T = 20
HW = H * W
P = 2
NPASS = T // P
NS = 16
CHUNK = 128
ZSL = P * HW // NS


def _sc_body(x_hbm, y_hbm, th_hbm, tl_hbm, v_hbm, z_hbm, out_hbm,
             xb, yb, thb, tlb, vb, ib0, wb0, ib1, wb1, stw, srd,
             plane, redsh, *, npad):
    ev_per_tile = npad // NS
    chunks = ev_per_tile // CHUNK
    c = lax.axis_index("c")
    s = lax.axis_index("s")
    ebase = c * npad + s * ev_per_tile

    def red_body(ch, carry):
        mn, mx = carry
        off = ebase + ch * CHUNK
        pltpu.sync_copy(th_hbm.at[pl.ds(off, CHUNK)], thb)
        pltpu.sync_copy(tl_hbm.at[pl.ds(off, CHUNK)], tlb)
        for v in range(CHUNK // 16):
            sl = pl.ds(v * 16, 16)
            mn = jnp.minimum(mn, thb[sl])
            mx = jnp.maximum(mx, tlb[sl])
        return mn, mx

    mn0 = jnp.full((16,), 2.0, jnp.float32)
    mx0 = jnp.full((16,), -1.0, jnp.float32)
    mn, mx = lax.fori_loop(0, chunks, red_body, (mn0, mx0))
    stw[pl.ds(0, 16)] = mn
    stw[pl.ds(16, 16)] = mx
    pltpu.sync_copy(stw, redsh.at[pl.ds(s * 32, 32)])
    plsc.subcore_barrier()
    pltpu.sync_copy(redsh, srd)
    tmin = jnp.float32(2.0)
    tmax = jnp.float32(-1.0)
    for i in range(NS):
        mnv = srd[pl.ds(i * 32, 16)]
        mxv = srd[pl.ds(i * 32 + 16, 16)]
        for j in range(16):
            tmin = jnp.minimum(tmin, mnv[j])
            tmax = jnp.maximum(tmax, mxv[j])
    rngv = jnp.maximum(jnp.broadcast_to(tmax - tmin, (16,)), 1e-9)
    inv = jnp.full((16,), T - 1.0, jnp.float32) / rngv

    def pass_body(pp, _):
        base = pp * P
        pltpu.sync_copy(z_hbm, plane.at[pl.ds(s * ZSL, ZSL)])
        plsc.subcore_barrier()

        def ch_body(ch, __):
            off = ebase + ch * CHUNK
            pltpu.sync_copy(x_hbm.at[pl.ds(off, CHUNK)], xb)
            pltpu.sync_copy(y_hbm.at[pl.ds(off, CHUNK)], yb)
            pltpu.sync_copy(th_hbm.at[pl.ds(off, CHUNK)], thb)
            pltpu.sync_copy(v_hbm.at[pl.ds(off, CHUNK)], vb)
            for v in range(CHUNK // 16):
                sl = pl.ds(v * 16, 16)
                xv = xb[sl]
                yv = yb[sl]
                tv = thb[sl]
                pv = vb[sl]
                xi = jnp.minimum(jnp.maximum(xv.astype(jnp.int32), 0), W - 1)
                yi = jnp.minimum(jnp.maximum(yv.astype(jnp.int32), 0), H - 1)
                tn = jnp.minimum(jnp.maximum((tv - tmin) * inv, 0.0),
                                 float(T - 1))
                t0 = tn.astype(jnp.int32)
                w1 = tn - t0.astype(jnp.float32)
                w0 = 1.0 - w1
                t1 = jnp.minimum(t0 + 1, T - 1)
                sp = yi * W + xi
                p0 = t0 - base
                p1 = t1 - base
                in0 = (p0 >= 0) & (p0 < P)
                in1 = (p1 >= 0) & (p1 < P)
                ib0[sl] = jnp.where(in0, p0 * HW + sp, 0)
                wb0[sl] = jnp.where(in0, pv * w0, 0.0)
                ib1[sl] = jnp.where(in1, p1 * HW + sp, 0)
                wb1[sl] = jnp.where(in1, pv * w1, 0.0)
            pltpu.sync_copy(wb0, plane.at[ib0], add=True)
            pltpu.sync_copy(wb1, plane.at[ib1], add=True)
            return 0

        lax.fori_loop(0, chunks, ch_body, 0)
        plsc.subcore_barrier()
        obase = c * (T * HW) + base * HW + s * ZSL
        pltpu.sync_copy(plane.at[pl.ds(s * ZSL, ZSL)],
                        out_hbm.at[pl.ds(obase, ZSL)])
        return 0

    lax.fori_loop(0, NPASS, pass_body, 0)


def kernel(eventBlock, eventCounts):
    b, n, _ = eventBlock.shape
    x = eventBlock[..., 0]
    y = eventBlock[..., 1]
    t = eventBlock[..., 2]
    p = eventBlock[..., 3]
    valid = jnp.arange(n, dtype=jnp.int32)[None, :] < eventCounts[:, None]
    th = jnp.where(valid, t, 2.0)
    tl = jnp.where(valid, t, -1.0)
    val = jnp.where(valid, p, 0.0)
    npad = ((n + NS * CHUNK - 1) // (NS * CHUNK)) * (NS * CHUNK)
    pad = npad - n

    def prep(a, cv):
        return jnp.pad(a, ((0, 0), (0, pad)), constant_values=cv).reshape(-1)

    xf = prep(x, 0.0)
    yf = prep(y, 0.0)
    thf = prep(th, 2.0)
    tlf = prep(tl, -1.0)
    vf = prep(val, 0.0)
    zeros = jnp.zeros((ZSL,), jnp.float32)

    mesh = plsc.VectorSubcoreMesh(core_axis_name="c", subcore_axis_name="s")
    k = functools.partial(
        pl.kernel,
        mesh=mesh,
        out_type=jax.ShapeDtypeStruct((b * T * HW,), jnp.float32),
        scratch_types=[
            pltpu.VMEM((CHUNK,), jnp.float32),
            pltpu.VMEM((CHUNK,), jnp.float32),
            pltpu.VMEM((CHUNK,), jnp.float32),
            pltpu.VMEM((CHUNK,), jnp.float32),
            pltpu.VMEM((CHUNK,), jnp.float32),
            pltpu.VMEM((CHUNK,), jnp.int32),
            pltpu.VMEM((CHUNK,), jnp.float32),
            pltpu.VMEM((CHUNK,), jnp.int32),
            pltpu.VMEM((CHUNK,), jnp.float32),
            pltpu.VMEM((32,), jnp.float32),
            pltpu.VMEM((NS * 32,), jnp.float32),
            pltpu.VMEM_SHARED((P * HW,), jnp.float32),
            pltpu.VMEM_SHARED((NS * 32,), jnp.float32),
        ],
    )(functools.partial(_sc_body, npad=npad))
    out = k(xf, yf, thf, tlf, vf, zeros)
    return out.reshape(b, T, H, W)


# speedup vs baseline: 1.0151x; 1.0151x over previous
import functools

import jax
import jax.numpy as jnp
from jax import lax
from jax.experimental import pallas as pl
from jax.experimental.pallas import tpu as pltpu
from jax.experimental.pallas import tpu_sc as plsc

H = 720
W = 1280
T = 20
HW = H * W
P = 2
NPASS = T // P
NS = 16
CHUNK = 128
LD = 1024
SUB = LD // CHUNK
ZSL = P * HW // NS


def _sc_body(x_hbm, y_hbm, th_hbm, tl_hbm, v_hbm, z_hbm, out_hbm, *scr,
             npad):
    xb, yb, thb, tlb, vb, stw, srd = scr[0:7]
    ib0s = scr[7:7 + SUB]
    wb0s = scr[7 + SUB:7 + 2 * SUB]
    ib1s = scr[7 + 2 * SUB:7 + 3 * SUB]
    wb1s = scr[7 + 3 * SUB:7 + 4 * SUB]
    plane = scr[7 + 4 * SUB]
    redsh = scr[8 + 4 * SUB]
    ldsem = scr[9 + 4 * SUB]
    scsem = scr[10 + 4 * SUB]

    ev_per_tile = npad // NS
    chunks = ev_per_tile // LD
    c = lax.axis_index("c")
    s = lax.axis_index("s")
    ebase = c * npad + s * ev_per_tile

    def red_body(ch, carry):
        mn, mx = carry
        off = ebase + ch * LD
        h1 = pltpu.async_copy(th_hbm.at[pl.ds(off, LD)], thb, ldsem)
        h2 = pltpu.async_copy(tl_hbm.at[pl.ds(off, LD)], tlb, ldsem)
        h1.wait()
        h2.wait()
        for v in range(LD // 16):
            sl = pl.ds(v * 16, 16)
            mn = jnp.minimum(mn, thb[sl])
            mx = jnp.maximum(mx, tlb[sl])
        return mn, mx

    mn0 = jnp.full((16,), 2.0, jnp.float32)
    mx0 = jnp.full((16,), -1.0, jnp.float32)
    mn, mx = lax.fori_loop(0, chunks, red_body, (mn0, mx0))
    stw[pl.ds(0, 16)] = mn
    stw[pl.ds(16, 16)] = mx
    pltpu.sync_copy(stw, redsh.at[pl.ds(s * 32, 32)])
    plsc.subcore_barrier()
    pltpu.sync_copy(redsh, srd)
    tmin = jnp.float32(2.0)
    tmax = jnp.float32(-1.0)
    for i in range(NS):
        mnv = srd[pl.ds(i * 32, 16)]
        mxv = srd[pl.ds(i * 32 + 16, 16)]
        for j in range(16):
            tmin = jnp.minimum(tmin, mnv[j])
            tmax = jnp.maximum(tmax, mxv[j])
    rngv = jnp.maximum(jnp.broadcast_to(tmax - tmin, (16,)), 1e-9)
    inv = jnp.full((16,), T - 1.0, jnp.float32) / rngv

    def pass_body(pp, _):
        base = pp * P
        pltpu.sync_copy(z_hbm, plane.at[pl.ds(s * ZSL, ZSL)])
        plsc.subcore_barrier()

        def ch_body(ch, __):
            off = ebase + ch * LD
            hs = [pltpu.async_copy(x_hbm.at[pl.ds(off, LD)], xb, ldsem),
                  pltpu.async_copy(y_hbm.at[pl.ds(off, LD)], yb, ldsem),
                  pltpu.async_copy(th_hbm.at[pl.ds(off, LD)], thb, ldsem),
                  pltpu.async_copy(v_hbm.at[pl.ds(off, LD)], vb, ldsem)]
            for h in hs:
                h.wait()
            shs = []
            for j in range(SUB):
                for v in range(CHUNK // 16):
                    sl = pl.ds(v * 16, 16)
                    ssl = pl.ds(j * CHUNK + v * 16, 16)
                    xv = xb[ssl]
                    yv = yb[ssl]
                    tv = thb[ssl]
                    pv = vb[ssl]
                    xi = jnp.minimum(jnp.maximum(xv.astype(jnp.int32), 0),
                                     W - 1)
                    yi = jnp.minimum(jnp.maximum(yv.astype(jnp.int32), 0),
                                     H - 1)
                    tn = jnp.minimum(jnp.maximum((tv - tmin) * inv, 0.0),
                                     float(T - 1))
                    t0 = tn.astype(jnp.int32)
                    w1 = tn - t0.astype(jnp.float32)
                    w0 = 1.0 - w1
                    t1 = jnp.minimum(t0 + 1, T - 1)
                    sp = yi * W + xi
                    p0 = t0 - base
                    p1 = t1 - base
                    in0 = (p0 >= 0) & (p0 < P)
                    in1 = (p1 >= 0) & (p1 < P)
                    ib0s[j][sl] = jnp.where(in0, p0 * HW + sp, 0)
                    wb0s[j][sl] = jnp.where(in0, pv * w0, 0.0)
                    ib1s[j][sl] = jnp.where(in1, p1 * HW + sp, 0)
                    wb1s[j][sl] = jnp.where(in1, pv * w1, 0.0)
                shs.append(pltpu.async_copy(wb0s[j], plane.at[ib0s[j]],
                                            scsem, add=True))
                shs.append(pltpu.async_copy(wb1s[j], plane.at[ib1s[j]],
                                            scsem, add=True))
            for h in shs:
                h.wait()
            return 0

        lax.fori_loop(0, chunks, ch_body, 0)
        plsc.subcore_barrier()
        obase = c * (T * HW) + base * HW + s * ZSL
        pltpu.sync_copy(plane.at[pl.ds(s * ZSL, ZSL)],
                        out_hbm.at[pl.ds(obase, ZSL)])
        return 0

    lax.fori_loop(0, NPASS, pass_body, 0)


def kernel(eventBlock, eventCounts):
    b, n, _ = eventBlock.shape
    x = eventBlock[..., 0]
    y = eventBlock[..., 1]
    t = eventBlock[..., 2]
    p = eventBlock[..., 3]
    valid = jnp.arange(n, dtype=jnp.int32)[None, :] < eventCounts[:, None]
    th = jnp.where(valid, t, 2.0)
    tl = jnp.where(valid, t, -1.0)
    val = jnp.where(valid, p, 0.0)
    npad = ((n + NS * LD - 1) // (NS * LD)) * (NS * LD)
    pad = npad - n

    def prep(a, cv):
        return jnp.pad(a, ((0, 0), (0, pad)), constant_values=cv).reshape(-1)

    xf = prep(x, 0.0)
    yf = prep(y, 0.0)
    thf = prep(th, 2.0)
    tlf = prep(tl, -1.0)
    vf = prep(val, 0.0)
    zeros = jnp.zeros((ZSL,), jnp.float32)

    scratch = [
        pltpu.VMEM((LD,), jnp.float32),
        pltpu.VMEM((LD,), jnp.float32),
        pltpu.VMEM((LD,), jnp.float32),
        pltpu.VMEM((LD,), jnp.float32),
        pltpu.VMEM((LD,), jnp.float32),
        pltpu.VMEM((32,), jnp.float32),
        pltpu.VMEM((NS * 32,), jnp.float32),
    ]
    scratch += [pltpu.VMEM((CHUNK,), jnp.int32) for _ in range(SUB)]
    scratch += [pltpu.VMEM((CHUNK,), jnp.float32) for _ in range(SUB)]
    scratch += [pltpu.VMEM((CHUNK,), jnp.int32) for _ in range(SUB)]
    scratch += [pltpu.VMEM((CHUNK,), jnp.float32) for _ in range(SUB)]
    scratch += [
        pltpu.VMEM_SHARED((P * HW,), jnp.float32),
        pltpu.VMEM_SHARED((NS * 32,), jnp.float32),
        pltpu.SemaphoreType.DMA,
        pltpu.SemaphoreType.DMA,
    ]

    mesh = plsc.VectorSubcoreMesh(core_axis_name="c", subcore_axis_name="s")
    k = functools.partial(
        pl.kernel,
        mesh=mesh,
        out_type=jax.ShapeDtypeStruct((b * T * HW,), jnp.float32),
        scratch_types=scratch,
    )(functools.partial(_sc_body, npad=npad))
    out = k(xf, yf, thf, tlf, vf, zeros)
    return out.reshape(b, T, H, W)
